# baseline (speedup 1.0000x reference)
import math

import jax
import jax.numpy as jnp
from jax.experimental import pallas as pl
from jax.experimental.pallas import tpu as pltpu
from jax.experimental.pallas import tpu_sc as plsc

S = 2048
H = 2048
HEADS = 16
HD = H // HEADS
FF = 5632
EPS = 1e-6

_PREC = jax.lax.Precision.HIGHEST


def _rms(x, w):
    v = jnp.mean(x * x, axis=-1, keepdims=True)
    return x * jax.lax.rsqrt(v + EPS) * w


_SPLIT = 8
_CH = H // _SPLIT


def _gather_embed(table8, idx8):
    mesh = plsc.VectorSubcoreMesh(core_axis_name="core", subcore_axis_name="subcore")
    W = 128
    N = idx8.shape[1]

    @pl.kernel(out_type=jax.ShapeDtypeStruct((N, _CH), table8.dtype), mesh=mesh)
    def k(x_hbm, i_hbm, o_hbm):
        def body(i_vmem, o_vmem):
            pltpu.sync_copy(x_hbm.at[i_vmem.at[0]], o_vmem)

        pltpu.emit_pipeline(
            body,
            grid=(N // W,),
            in_specs=[pl.BlockSpec((1, W), lambda i: (0, i))],
            out_specs=[pl.BlockSpec((W, _CH), lambda i: (i, 0))],
            core_axis_name=("core", "subcore"),
            dimension_semantics=(pltpu.PARALLEL,),
        )(i_hbm, o_hbm)

    return k(table8, idx8)


def _fc(e, hs, fc_w, b, interpret=False):
    BS = 256
    KB = 512
    nk = (2 * H) // KB
    ns = S // BS

    def body(e_ref, h_ref, w_ref, b_ref, o_ref, acc_ref):
        kk = pl.program_id(0)
        i = pl.program_id(1)
        x = jnp.where(kk < nk // 2, e_ref[...], h_ref[...])
        p = jnp.dot(x, w_ref[...], preferred_element_type=jnp.float32,
                    precision=_PREC)
        sl = pl.ds(i * BS, BS)

        @pl.when(kk == 0)
        def _():
            acc_ref[sl, :] = p

        @pl.when(kk > 0)
        def _():
            acc_ref[sl, :] += p

        @pl.when(kk == nk - 1)
        def _():
            o_ref[...] = acc_ref[sl, :] + b_ref[...]

    return pl.pallas_call(
        body,
        grid=(nk, ns),
        in_specs=[
            pl.BlockSpec((BS, KB), lambda kk, i: (i, jnp.minimum(kk, nk // 2 - 1))),
            pl.BlockSpec((BS, KB), lambda kk, i: (i, jnp.maximum(kk - nk // 2, 0))),
            pl.BlockSpec((KB, H), lambda kk, i: (kk, 0)),
            pl.BlockSpec((1, H), lambda kk, i: (0, 0)),
        ],
        out_specs=pl.BlockSpec(
            (BS, H), lambda kk, i: (jnp.where(kk == nk - 1, i, 0), 0)),
        out_shape=jax.ShapeDtypeStruct((S, H), jnp.float32),
        scratch_shapes=[pltpu.VMEM((S, H), jnp.float32)],
        interpret=interpret,
    )(e, hs, fc_w, b)


def _qkv(h, wqkv, ln1, cos2, sin2, interpret=False):
    BS = 256

    def body(h_ref, w_ref, ln_ref, c_ref, s_ref, o_ref):
        j = pl.program_id(0)
        x = _rms(h_ref[...], ln_ref[...])
        p = jnp.dot(x, w_ref[0], preferred_element_type=jnp.float32, precision=_PREC)
        pr = p.reshape(BS, HEADS, HD)
        c = c_ref[...][:, None, :]
        s = s_ref[...][:, None, :]
        x1 = pr[..., : HD // 2]
        x2 = pr[..., HD // 2:]
        rot = jnp.concatenate([-x2, x1], axis=-1)
        rotated = pr * c + rot * s
        o = jnp.where(j < 2, rotated, pr)
        o_ref[0] = o.reshape(BS, H)

    return pl.pallas_call(
        body,
        grid=(3, S // BS),
        in_specs=[
            pl.BlockSpec((BS, H), lambda j, i: (i, 0)),
            pl.BlockSpec((1, H, H), lambda j, i: (j, 0, 0)),
            pl.BlockSpec((1, H), lambda j, i: (0, 0)),
            pl.BlockSpec((BS, HD), lambda j, i: (i, 0)),
            pl.BlockSpec((BS, HD), lambda j, i: (i, 0)),
        ],
        out_specs=pl.BlockSpec((1, BS, H), lambda j, i: (j, i, 0)),
        out_shape=jax.ShapeDtypeStruct((3, S, H), jnp.float32),
        interpret=interpret,
    )(h, wqkv, ln1, cos2, sin2)


def _attention(q, k, v, interpret=False):
    BL = 512
    nq = S // BL
    nk = S // BL
    scale = 1.0 / math.sqrt(HD)

    def body(q_ref, k_ref, v_ref, o_ref, acc_ref, m_ref, l_ref):
        i = pl.program_id(1)
        kk = pl.program_id(2)

        @pl.when(kk == 0)
        def _():
            acc_ref[...] = jnp.zeros_like(acc_ref)
            m_ref[...] = jnp.full_like(m_ref, -1e30)
            l_ref[...] = jnp.zeros_like(l_ref)

        @pl.when(kk <= i)
        def _():
            qb = q_ref[...] * scale
            s = jax.lax.dot_general(
                qb, k_ref[...], (((1,), (1,)), ((), ())),
                preferred_element_type=jnp.float32, precision=_PREC)
            rows = i * BL + jax.lax.broadcasted_iota(jnp.int32, (BL, BL), 0)
            cols = kk * BL + jax.lax.broadcasted_iota(jnp.int32, (BL, BL), 1)
            s = jnp.where(rows >= cols, s, -1e30)
            m_prev = m_ref[...]
            m_new = jnp.maximum(m_prev, jnp.max(s, axis=-1, keepdims=True))
            alpha = jnp.exp(m_prev - m_new)
            p = jnp.exp(s - m_new)
            l_ref[...] = l_ref[...] * alpha + jnp.sum(p, axis=-1, keepdims=True)
            acc_ref[...] = acc_ref[...] * alpha + jnp.dot(
                p, v_ref[...], preferred_element_type=jnp.float32, precision=_PREC)
            m_ref[...] = m_new

        @pl.when(kk == nk - 1)
        def _():
            o_ref[...] = acc_ref[...] / l_ref[...]

    return pl.pallas_call(
        body,
        grid=(HEADS, nq, nk),
        in_specs=[
            pl.BlockSpec((BL, HD), lambda h, i, kk: (i, h)),
            pl.BlockSpec((BL, HD), lambda h, i, kk: (jnp.minimum(kk, i), h)),
            pl.BlockSpec((BL, HD), lambda h, i, kk: (jnp.minimum(kk, i), h)),
        ],
        out_specs=pl.BlockSpec((BL, HD), lambda h, i, kk: (i, h)),
        out_shape=jax.ShapeDtypeStruct((S, H), jnp.float32),
        scratch_shapes=[
            pltpu.VMEM((BL, HD), jnp.float32),
            pltpu.VMEM((BL, 1), jnp.float32),
            pltpu.VMEM((BL, 1), jnp.float32),
        ],
        interpret=interpret,
    )(q, k, v)


def _wo(h, o_attn, wo, ln2, interpret=False):
    BS = 128

    def body(h_ref, oa_ref, w_ref, ln_ref, h2_ref, x2_ref):
        h2 = h_ref[...] + jnp.dot(oa_ref[...], w_ref[...],
                                  preferred_element_type=jnp.float32, precision=_PREC)
        h2_ref[...] = h2
        x2_ref[...] = _rms(h2, ln_ref[...])

    return pl.pallas_call(
        body,
        grid=(S // BS,),
        in_specs=[
            pl.BlockSpec((BS, H), lambda i: (i, 0)),
            pl.BlockSpec((BS, H), lambda i: (i, 0)),
            pl.BlockSpec((H, H), lambda i: (0, 0)),
            pl.BlockSpec((1, H), lambda i: (0, 0)),
        ],
        out_specs=[
            pl.BlockSpec((BS, H), lambda i: (i, 0)),
            pl.BlockSpec((BS, H), lambda i: (i, 0)),
        ],
        out_shape=[
            jax.ShapeDtypeStruct((S, H), jnp.float32),
            jax.ShapeDtypeStruct((S, H), jnp.float32),
        ],
        interpret=interpret,
    )(h, o_attn, wo, ln2)


def _gateup(x2, wg, wu, interpret=False):
    FFB = 256

    def body(x_ref, wg_ref, wu_ref, a_ref):
        x = x_ref[...]
        g = jnp.dot(x, wg_ref[...], preferred_element_type=jnp.float32, precision=_PREC)
        u = jnp.dot(x, wu_ref[...], preferred_element_type=jnp.float32, precision=_PREC)
        a_ref[...] = jax.nn.silu(g) * u

    return pl.pallas_call(
        body,
        grid=(FF // FFB,),
        in_specs=[
            pl.BlockSpec((S, H), lambda i: (0, 0)),
            pl.BlockSpec((H, FFB), lambda i: (0, i)),
            pl.BlockSpec((H, FFB), lambda i: (0, i)),
        ],
        out_specs=pl.BlockSpec((S, FFB), lambda i: (0, i)),
        out_shape=jax.ShapeDtypeStruct((S, FF), jnp.float32),
        interpret=interpret,
    )(x2, wg, wu)


def _down(act, wd, interpret=False):
    FFB = 256
    nff = FF // FFB

    def body(a_ref, wd_ref, o_ref):
        i = pl.program_id(0)
        p = jnp.dot(a_ref[...], wd_ref[...], preferred_element_type=jnp.float32,
                    precision=_PREC)

        @pl.when(i == 0)
        def _():
            o_ref[...] = p

        @pl.when(i > 0)
        def _():
            o_ref[...] = o_ref[...] + p

    return pl.pallas_call(
        body,
        grid=(nff,),
        in_specs=[
            pl.BlockSpec((S, FFB), lambda i: (0, i)),
            pl.BlockSpec((FFB, H), lambda i: (i, 0)),
        ],
        out_specs=pl.BlockSpec((S, H), lambda i: (0, 0)),
        out_shape=jax.ShapeDtypeStruct((S, H), jnp.float32),
        interpret=interpret,
    )(act, wd)


def _final(h2, mlp, normw, interpret=False):
    BS = 256

    def body(h2_ref, m_ref, nw_ref, o_ref):
        o_ref[...] = _rms(h2_ref[...] + m_ref[...], nw_ref[...])

    return pl.pallas_call(
        body,
        grid=(S // BS,),
        in_specs=[
            pl.BlockSpec((BS, H), lambda i: (i, 0)),
            pl.BlockSpec((BS, H), lambda i: (i, 0)),
            pl.BlockSpec((1, H), lambda i: (0, 0)),
        ],
        out_specs=pl.BlockSpec((BS, H), lambda i: (i, 0)),
        out_shape=jax.ShapeDtypeStruct((S, H), jnp.float32),
        interpret=interpret,
    )(h2, mlp, normw)


def _rotary_tables():
    half = HD // 2
    inv = 1.0 / (10000.0 ** (jnp.arange(0, half, dtype=jnp.float32) / half))
    pos = jnp.arange(S, dtype=jnp.float32)
    freqs = pos[:, None] * inv[None, :]
    cos2 = jnp.concatenate([jnp.cos(freqs), jnp.cos(freqs)], axis=-1)
    sin2 = jnp.concatenate([jnp.sin(freqs), jnp.sin(freqs)], axis=-1)
    return cos2, sin2


def kernel(hidden_states, input_ids, embed_table, fc_w, fc_b, wq, wk, wv, wo,
           w_gate, w_up, w_down, ln1_w, ln2_w, norm_w):
    hs = hidden_states[0]
    ids = input_ids.astype(jnp.int32).reshape(S)
    table8 = embed_table.reshape(-1, _CH)
    idx8 = (ids[:, None] * _SPLIT
            + jnp.arange(_SPLIT, dtype=jnp.int32)[None, :]).reshape(1, S * _SPLIT)
    e = _gather_embed(table8, idx8).reshape(S, H)
    h = _fc(e, hs, fc_w, fc_b.reshape(1, H))
    cos2, sin2 = _rotary_tables()
    wqkv = jnp.stack([wq, wk, wv])
    qkv = _qkv(h, wqkv, ln1_w.reshape(1, H), cos2, sin2)
    o = _attention(qkv[0], qkv[1], qkv[2])
    h2, x2 = _wo(h, o, wo, ln2_w.reshape(1, H))
    act = _gateup(x2, w_gate, w_up)
    mlp = _down(act, w_down)
    out = _final(h2, mlp, norm_w.reshape(1, H))
    return out[None]


# speedup vs baseline: 2.2652x; 2.2652x over previous
import math

import jax
import jax.numpy as jnp
from jax.experimental import pallas as pl
from jax.experimental.pallas import tpu as pltpu
from jax.experimental.pallas import tpu_sc as plsc

S = 2048
H = 2048
HEADS = 16
HD = H // HEADS
FF = 5632
EPS = 1e-6

_PREC = jax.lax.Precision.DEFAULT


def _rms(x, w):
    v = jnp.mean(x * x, axis=-1, keepdims=True)
    return x * jax.lax.rsqrt(v + EPS) * w


_SPLIT = 8
_CH = H // _SPLIT


def _gather_embed(table8, idx8):
    mesh = plsc.VectorSubcoreMesh(core_axis_name="core", subcore_axis_name="subcore")
    W = 128
    N = idx8.shape[1]

    @pl.kernel(out_type=jax.ShapeDtypeStruct((N, _CH), table8.dtype), mesh=mesh)
    def k(x_hbm, i_hbm, o_hbm):
        def body(i_vmem, o_vmem):
            pltpu.sync_copy(x_hbm.at[i_vmem.at[0]], o_vmem)

        pltpu.emit_pipeline(
            body,
            grid=(N // W,),
            in_specs=[pl.BlockSpec((1, W), lambda i: (0, i))],
            out_specs=[pl.BlockSpec((W, _CH), lambda i: (i, 0))],
            core_axis_name=("core", "subcore"),
            dimension_semantics=(pltpu.PARALLEL,),
        )(i_hbm, o_hbm)

    return k(table8, idx8)


def _fc(e, hs, fc_w, b, interpret=False):
    BS = 256
    KB = 512
    nk = (2 * H) // KB
    ns = S // BS

    def body(e_ref, h_ref, w_ref, b_ref, o_ref, acc_ref):
        kk = pl.program_id(0)
        i = pl.program_id(1)
        x = jnp.where(kk < nk // 2, e_ref[...], h_ref[...])
        p = jnp.dot(x, w_ref[...], preferred_element_type=jnp.float32,
                    precision=_PREC)
        sl = pl.ds(i * BS, BS)

        @pl.when(kk == 0)
        def _():
            acc_ref[sl, :] = p

        @pl.when(kk > 0)
        def _():
            acc_ref[sl, :] += p

        @pl.when(kk == nk - 1)
        def _():
            o_ref[...] = acc_ref[sl, :] + b_ref[...]

    return pl.pallas_call(
        body,
        grid=(nk, ns),
        in_specs=[
            pl.BlockSpec((BS, KB), lambda kk, i: (i, jnp.minimum(kk, nk // 2 - 1))),
            pl.BlockSpec((BS, KB), lambda kk, i: (i, jnp.maximum(kk - nk // 2, 0))),
            pl.BlockSpec((KB, H), lambda kk, i: (kk, 0)),
            pl.BlockSpec((1, H), lambda kk, i: (0, 0)),
        ],
        out_specs=pl.BlockSpec(
            (BS, H), lambda kk, i: (jnp.where(kk == nk - 1, i, 0), 0)),
        out_shape=jax.ShapeDtypeStruct((S, H), jnp.float32),
        scratch_shapes=[pltpu.VMEM((S, H), jnp.float32)],
        interpret=interpret,
    )(e, hs, fc_w, b)


def _qkv(h, wqkv, ln1, cos2, sin2, interpret=False):
    BS = 256

    def body(h_ref, w_ref, ln_ref, c_ref, s_ref, o_ref):
        j = pl.program_id(0)
        x = _rms(h_ref[...], ln_ref[...])
        p = jnp.dot(x, w_ref[0], preferred_element_type=jnp.float32, precision=_PREC)
        pr = p.reshape(BS, HEADS, HD)
        c = c_ref[...][:, None, :]
        s = s_ref[...][:, None, :]
        x1 = pr[..., : HD // 2]
        x2 = pr[..., HD // 2:]
        rot = jnp.concatenate([-x2, x1], axis=-1)
        rotated = pr * c + rot * s
        o = jnp.where(j < 2, rotated, pr)
        o_ref[0] = o.reshape(BS, H)

    return pl.pallas_call(
        body,
        grid=(3, S // BS),
        in_specs=[
            pl.BlockSpec((BS, H), lambda j, i: (i, 0)),
            pl.BlockSpec((1, H, H), lambda j, i: (j, 0, 0)),
            pl.BlockSpec((1, H), lambda j, i: (0, 0)),
            pl.BlockSpec((BS, HD), lambda j, i: (i, 0)),
            pl.BlockSpec((BS, HD), lambda j, i: (i, 0)),
        ],
        out_specs=pl.BlockSpec((1, BS, H), lambda j, i: (j, i, 0)),
        out_shape=jax.ShapeDtypeStruct((3, S, H), jnp.float32),
        interpret=interpret,
    )(h, wqkv, ln1, cos2, sin2)


def _attention(q, k, v, interpret=False):
    BL = 512
    nq = S // BL
    nk = S // BL
    scale = 1.0 / math.sqrt(HD)

    def body(q_ref, k_ref, v_ref, o_ref, acc_ref, m_ref, l_ref):
        i = pl.program_id(1)
        kk = pl.program_id(2)

        @pl.when(kk == 0)
        def _():
            acc_ref[...] = jnp.zeros_like(acc_ref)
            m_ref[...] = jnp.full_like(m_ref, -1e30)
            l_ref[...] = jnp.zeros_like(l_ref)

        @pl.when(kk <= i)
        def _():
            qb = q_ref[...] * scale
            s = jax.lax.dot_general(
                qb, k_ref[...], (((1,), (1,)), ((), ())),
                preferred_element_type=jnp.float32, precision=_PREC)
            rows = i * BL + jax.lax.broadcasted_iota(jnp.int32, (BL, BL), 0)
            cols = kk * BL + jax.lax.broadcasted_iota(jnp.int32, (BL, BL), 1)
            s = jnp.where(rows >= cols, s, -1e30)
            m_prev = m_ref[...]
            m_new = jnp.maximum(m_prev, jnp.max(s, axis=-1, keepdims=True))
            alpha = jnp.exp(m_prev - m_new)
            p = jnp.exp(s - m_new)
            l_ref[...] = l_ref[...] * alpha + jnp.sum(p, axis=-1, keepdims=True)
            acc_ref[...] = acc_ref[...] * alpha + jnp.dot(
                p, v_ref[...], preferred_element_type=jnp.float32, precision=_PREC)
            m_ref[...] = m_new

        @pl.when(kk == nk - 1)
        def _():
            o_ref[...] = acc_ref[...] / l_ref[...]

    return pl.pallas_call(
        body,
        grid=(HEADS, nq, nk),
        in_specs=[
            pl.BlockSpec((BL, HD), lambda h, i, kk: (i, h)),
            pl.BlockSpec((BL, HD), lambda h, i, kk: (jnp.minimum(kk, i), h)),
            pl.BlockSpec((BL, HD), lambda h, i, kk: (jnp.minimum(kk, i), h)),
        ],
        out_specs=pl.BlockSpec((BL, HD), lambda h, i, kk: (i, h)),
        out_shape=jax.ShapeDtypeStruct((S, H), jnp.float32),
        scratch_shapes=[
            pltpu.VMEM((BL, HD), jnp.float32),
            pltpu.VMEM((BL, 1), jnp.float32),
            pltpu.VMEM((BL, 1), jnp.float32),
        ],
        interpret=interpret,
    )(q, k, v)


def _wo(h, o_attn, wo, ln2, interpret=False):
    BS = 128

    def body(h_ref, oa_ref, w_ref, ln_ref, h2_ref, x2_ref):
        h2 = h_ref[...] + jnp.dot(oa_ref[...], w_ref[...],
                                  preferred_element_type=jnp.float32, precision=_PREC)
        h2_ref[...] = h2
        x2_ref[...] = _rms(h2, ln_ref[...])

    return pl.pallas_call(
        body,
        grid=(S // BS,),
        in_specs=[
            pl.BlockSpec((BS, H), lambda i: (i, 0)),
            pl.BlockSpec((BS, H), lambda i: (i, 0)),
            pl.BlockSpec((H, H), lambda i: (0, 0)),
            pl.BlockSpec((1, H), lambda i: (0, 0)),
        ],
        out_specs=[
            pl.BlockSpec((BS, H), lambda i: (i, 0)),
            pl.BlockSpec((BS, H), lambda i: (i, 0)),
        ],
        out_shape=[
            jax.ShapeDtypeStruct((S, H), jnp.float32),
            jax.ShapeDtypeStruct((S, H), jnp.float32),
        ],
        interpret=interpret,
    )(h, o_attn, wo, ln2)


def _gateup(x2, wg, wu, interpret=False):
    FFB = 256

    def body(x_ref, wg_ref, wu_ref, a_ref):
        x = x_ref[...]
        g = jnp.dot(x, wg_ref[...], preferred_element_type=jnp.float32, precision=_PREC)
        u = jnp.dot(x, wu_ref[...], preferred_element_type=jnp.float32, precision=_PREC)
        a_ref[...] = jax.nn.silu(g) * u

    return pl.pallas_call(
        body,
        grid=(FF // FFB,),
        in_specs=[
            pl.BlockSpec((S, H), lambda i: (0, 0)),
            pl.BlockSpec((H, FFB), lambda i: (0, i)),
            pl.BlockSpec((H, FFB), lambda i: (0, i)),
        ],
        out_specs=pl.BlockSpec((S, FFB), lambda i: (0, i)),
        out_shape=jax.ShapeDtypeStruct((S, FF), jnp.float32),
        interpret=interpret,
    )(x2, wg, wu)


def _down(act, wd, interpret=False):
    FFB = 256
    nff = FF // FFB

    def body(a_ref, wd_ref, o_ref):
        i = pl.program_id(0)
        p = jnp.dot(a_ref[...], wd_ref[...], preferred_element_type=jnp.float32,
                    precision=_PREC)

        @pl.when(i == 0)
        def _():
            o_ref[...] = p

        @pl.when(i > 0)
        def _():
            o_ref[...] = o_ref[...] + p

    return pl.pallas_call(
        body,
        grid=(nff,),
        in_specs=[
            pl.BlockSpec((S, FFB), lambda i: (0, i)),
            pl.BlockSpec((FFB, H), lambda i: (i, 0)),
        ],
        out_specs=pl.BlockSpec((S, H), lambda i: (0, 0)),
        out_shape=jax.ShapeDtypeStruct((S, H), jnp.float32),
        interpret=interpret,
    )(act, wd)


def _final(h2, mlp, normw, interpret=False):
    BS = 256

    def body(h2_ref, m_ref, nw_ref, o_ref):
        o_ref[...] = _rms(h2_ref[...] + m_ref[...], nw_ref[...])

    return pl.pallas_call(
        body,
        grid=(S // BS,),
        in_specs=[
            pl.BlockSpec((BS, H), lambda i: (i, 0)),
            pl.BlockSpec((BS, H), lambda i: (i, 0)),
            pl.BlockSpec((1, H), lambda i: (0, 0)),
        ],
        out_specs=pl.BlockSpec((BS, H), lambda i: (i, 0)),
        out_shape=jax.ShapeDtypeStruct((S, H), jnp.float32),
        interpret=interpret,
    )(h2, mlp, normw)


def _rotary_tables():
    half = HD // 2
    inv = 1.0 / (10000.0 ** (jnp.arange(0, half, dtype=jnp.float32) / half))
    pos = jnp.arange(S, dtype=jnp.float32)
    freqs = pos[:, None] * inv[None, :]
    cos2 = jnp.concatenate([jnp.cos(freqs), jnp.cos(freqs)], axis=-1)
    sin2 = jnp.concatenate([jnp.sin(freqs), jnp.sin(freqs)], axis=-1)
    return cos2, sin2


def kernel(hidden_states, input_ids, embed_table, fc_w, fc_b, wq, wk, wv, wo,
           w_gate, w_up, w_down, ln1_w, ln2_w, norm_w):
    hs = hidden_states[0]
    ids = input_ids.astype(jnp.int32).reshape(S)
    table8 = embed_table.reshape(-1, _CH)
    idx8 = (ids[:, None] * _SPLIT
            + jnp.arange(_SPLIT, dtype=jnp.int32)[None, :]).reshape(1, S * _SPLIT)
    e = _gather_embed(table8, idx8).reshape(S, H)
    h = _fc(e, hs, fc_w, fc_b.reshape(1, H))
    cos2, sin2 = _rotary_tables()
    wqkv = jnp.stack([wq, wk, wv])
    qkv = _qkv(h, wqkv, ln1_w.reshape(1, H), cos2, sin2)
    o = _attention(qkv[0], qkv[1], qkv[2])
    h2, x2 = _wo(h, o, wo, ln2_w.reshape(1, H))
    act = _gateup(x2, w_gate, w_up)
    mlp = _down(act, w_down)
    out = _final(h2, mlp, norm_w.reshape(1, H))
    return out[None]
